# baseline (speedup 1.0000x reference)
import functools
import math

import jax
import jax.numpy as jnp
import numpy as np
from jax import lax
from jax.experimental import pallas as pl
from jax.experimental.pallas import tpu as pltpu
from jax.experimental.pallas import tpu_sc as plsc

_CHUNK = 64


@functools.lru_cache(maxsize=None)
def _make_sc_gather(vocab: int, emb_dim: int, n_idx: int, tok_off: int,
                    n_tok: int):
    info = plsc.get_sparse_core_info()
    nw = info.num_cores * info.num_subcores
    assert n_tok % (nw * _CHUNK) == 0
    chunks_per_w = n_tok // (nw * _CHUNK)

    mesh = plsc.VectorSubcoreMesh(core_axis_name="c", subcore_axis_name="s")

    tok_per_w = chunks_per_w * _CHUNK

    @functools.partial(
        pl.kernel,
        out_type=jax.ShapeDtypeStruct((n_tok, emb_dim), jnp.float32),
        mesh=mesh,
        scratch_types=[
            pltpu.VMEM((tok_per_w,), jnp.int32),
            pltpu.VMEM((tok_per_w, emb_dim), jnp.float32),
            pltpu.SemaphoreType.DMA,
        ],
    )
    def gather_kernel(table_hbm, idx_hbm, out_hbm, idx_v, rows_v, sem):
        wid = lax.axis_index("s") * info.num_cores + lax.axis_index("c")
        base = wid * tok_per_w
        pltpu.sync_copy(idx_hbm.at[pl.ds(tok_off + base, tok_per_w)], idx_v)
        copies = [
            pltpu.async_copy(
                table_hbm.at[idx_v.at[pl.ds(j * _CHUNK, _CHUNK)]],
                rows_v.at[pl.ds(j * _CHUNK, _CHUNK)], sem)
            for j in range(chunks_per_w)
        ]
        for c in copies:
            c.wait()
        pltpu.sync_copy(rows_v, out_hbm.at[pl.ds(base, tok_per_w)])

    return gather_kernel


def _emb_step(xv, g, pe_row, r0, *, scale):
    b_dim, e_dim = g.shape
    z = (xv == 0).astype(jnp.float32)
    z3 = lax.broadcast_in_dim(z, (8, b_dim // 8, e_dim), (0, 1))
    r3 = lax.broadcast_in_dim(r0, (8, b_dim // 8, e_dim), (2,))
    g3 = g.reshape(8, b_dim // 8, e_dim)
    emb = (g3 - z3 * r3).reshape(b_dim, e_dim) * scale + pe_row
    return emb.astype(jnp.bfloat16)


_N_HALF = 2


def _tc_body(x_ref, g_hbm, pe_ref, w_ref, b_ref, t_hbm, o_hbm,
             g_buf, o_buf, r0_buf, g_sem, o_sem, r_sem, *,
             scale, s_off, s_cnt):
    d_model = o_buf.shape[2]
    dh = d_model // _N_HALF

    def g_copy(i):
        return pltpu.make_async_copy(
            g_hbm.at[i], g_buf.at[i % 2], g_sem.at[i % 2])

    def o_copy(i, h):
        return pltpu.make_async_copy(
            o_buf.at[i % 2, slice(None), pl.ds(h * dh, dh)],
            o_hbm.at[s_off + i, slice(None), pl.ds(h * dh, dh)],
            o_sem.at[i % 2, h])

    r_copy = pltpu.make_async_copy(t_hbm.at[pl.ds(0, 8)], r0_buf, r_sem)
    r_copy.start()
    g_copy(0).start()
    g_copy(1).start()
    r_copy.wait()
    r0 = r0_buf[0]
    wb = w_ref[...].astype(jnp.bfloat16)

    for i in range(s_cnt):
        g_copy(i).wait()
        if i >= 2:
            for h in range(_N_HALF):
                o_copy(i - 2, h).wait()
        embb = _emb_step(x_ref[s_off + i], g_buf[i % 2], pe_ref[s_off + i],
                         r0, scale=scale)
        for h in range(_N_HALF):
            o_buf[i % 2, :, h * dh:(h + 1) * dh] = (
                jnp.dot(embb, wb[:, h * dh:(h + 1) * dh],
                        preferred_element_type=jnp.float32)
                + b_ref[:, h * dh:(h + 1) * dh]
            )
            o_copy(i, h).start()
        if i + 2 < s_cnt:
            g_copy(i + 2).start()
    for i in (s_cnt - 2, s_cnt - 1):
        for h in range(_N_HALF):
            o_copy(i, h).wait()


def _positional_encoding(seq_len, d):
    position = np.arange(seq_len, dtype=np.float32)[:, None]
    div_term = np.exp(
        np.arange(0, d, 2, dtype=np.float32) * (-np.log(10000.0) / d)
    )
    pe = np.zeros((seq_len, d), dtype=np.float32)
    pe[:, 0::2] = np.sin(position * div_term)
    pe[:, 1::2] = np.cos(position * div_term)
    return jnp.asarray(pe)


def _tc_body_acc(x_ref, g_hbm, pe_ref, w_ref, b_ref, t_hbm, prev_ref,
                 o_hbm, *scratch, scale, s_off, s_cnt):
    del prev_ref
    _tc_body(x_ref, g_hbm, pe_ref, w_ref, b_ref, t_hbm, o_hbm, *scratch,
             scale=scale, s_off=s_off, s_cnt=s_cnt)


_SLICES = (6, 12, 12, 12, 8)


def kernel(x, x_pad_mask, emb_table, proj_w, proj_b):
    seq, batch = x.shape
    vocab, emb_dim = emb_table.shape
    d_model = proj_w.shape[1]
    n_tok = seq * batch
    scale = math.sqrt(float(emb_dim))
    assert seq == sum(_SLICES)

    x = x.astype(jnp.int32)
    idx = x.reshape(n_tok)

    gathered = []
    off = 0
    for s_cnt in _SLICES:
        gathered.append(
            _make_sc_gather(vocab, emb_dim, n_tok, off * batch,
                            s_cnt * batch)(emb_table, idx)
        )
        off += s_cnt

    pe = _positional_encoding(seq, emb_dim).reshape(seq, 1, emb_dim)
    x3 = x.reshape(seq, batch // 128, 128)
    b2 = proj_b.reshape(1, d_model)

    out_shape = jax.ShapeDtypeStruct((seq, batch, d_model), jnp.float32)
    vmem = pl.BlockSpec(memory_space=pltpu.MemorySpace.VMEM)
    hbm = pl.BlockSpec(memory_space=pltpu.MemorySpace.HBM)
    scratch_shapes = [
        pltpu.VMEM((2, batch, emb_dim), jnp.float32),
        pltpu.VMEM((2, batch, d_model), jnp.float32),
        pltpu.VMEM((8, emb_dim), jnp.float32),
        pltpu.SemaphoreType.DMA((2,)),
        pltpu.SemaphoreType.DMA((2, _N_HALF)),
        pltpu.SemaphoreType.DMA,
    ]

    out = None
    off = 0
    for k, s_cnt in enumerate(_SLICES):
        g3 = gathered[k].reshape(s_cnt, batch, emb_dim)
        if out is None:
            out = pl.pallas_call(
                functools.partial(_tc_body, scale=scale, s_off=off,
                                  s_cnt=s_cnt),
                in_specs=[vmem, hbm, vmem, vmem, vmem, hbm],
                out_specs=hbm,
                out_shape=out_shape,
                scratch_shapes=scratch_shapes,
            )(x3, g3, pe, proj_w, b2, emb_table)
        else:
            out = pl.pallas_call(
                functools.partial(_tc_body_acc, scale=scale, s_off=off,
                                  s_cnt=s_cnt),
                in_specs=[vmem, hbm, vmem, vmem, vmem, hbm, hbm],
                out_specs=hbm,
                out_shape=out_shape,
                scratch_shapes=scratch_shapes,
                input_output_aliases={6: 0},
            )(x3, g3, pe, proj_w, b2, emb_table, out)
        off += s_cnt
    return out


# speedup vs baseline: 1.0177x; 1.0177x over previous
import functools
import math

import jax
import jax.numpy as jnp
import numpy as np
from jax import lax
from jax.experimental import pallas as pl
from jax.experimental.pallas import tpu as pltpu
from jax.experimental.pallas import tpu_sc as plsc

_CHUNK = 64


@functools.lru_cache(maxsize=None)
def _make_sc_gather(vocab: int, emb_dim: int, n_idx: int, tok_off: int,
                    n_tok: int):
    info = plsc.get_sparse_core_info()
    nw = info.num_cores * info.num_subcores
    assert n_tok % (nw * _CHUNK) == 0
    chunks_per_w = n_tok // (nw * _CHUNK)

    mesh = plsc.VectorSubcoreMesh(core_axis_name="c", subcore_axis_name="s")

    tok_per_w = chunks_per_w * _CHUNK

    @functools.partial(
        pl.kernel,
        out_type=jax.ShapeDtypeStruct((n_tok, emb_dim), jnp.float32),
        mesh=mesh,
        scratch_types=[
            pltpu.VMEM((tok_per_w,), jnp.int32),
            pltpu.VMEM((tok_per_w, emb_dim), jnp.float32),
            pltpu.SemaphoreType.DMA,
        ],
    )
    def gather_kernel(table_hbm, idx_hbm, out_hbm, idx_v, rows_v, sem):
        wid = lax.axis_index("s") * info.num_cores + lax.axis_index("c")
        base = wid * tok_per_w
        pltpu.sync_copy(idx_hbm.at[pl.ds(tok_off + base, tok_per_w)], idx_v)
        copies = [
            pltpu.async_copy(
                table_hbm.at[idx_v.at[pl.ds(j * _CHUNK, _CHUNK)]],
                rows_v.at[pl.ds(j * _CHUNK, _CHUNK)], sem)
            for j in range(chunks_per_w)
        ]
        for c in copies:
            c.wait()
        pltpu.sync_copy(rows_v, out_hbm.at[pl.ds(base, tok_per_w)])

    return gather_kernel


def _emb_step(xv, g, pe_row, r0, *, scale):
    b_dim, e_dim = g.shape
    z = (xv == 0).astype(jnp.float32)
    z3 = lax.broadcast_in_dim(z, (8, b_dim // 8, e_dim), (0, 1))
    r3 = lax.broadcast_in_dim(r0, (8, b_dim // 8, e_dim), (2,))
    g3 = g.reshape(8, b_dim // 8, e_dim)
    emb = (g3 - z3 * r3).reshape(b_dim, e_dim) * scale + pe_row
    return emb.astype(jnp.bfloat16)


_N_HALF = 2


def _tc_body(x_ref, g_hbm, pe_ref, w_ref, b_ref, t_hbm, o_hbm,
             g_buf, o_buf, r0_buf, g_sem, o_sem, r_sem, *,
             scale, s_off, s_cnt):
    d_model = o_buf.shape[2]
    dh = d_model // _N_HALF

    def g_copy(i):
        return pltpu.make_async_copy(
            g_hbm.at[i], g_buf.at[i % 2], g_sem.at[i % 2])

    def o_copy(i, h):
        return pltpu.make_async_copy(
            o_buf.at[i % 2, slice(None), pl.ds(h * dh, dh)],
            o_hbm.at[s_off + i, slice(None), pl.ds(h * dh, dh)],
            o_sem.at[i % 2, h])

    r_copy = pltpu.make_async_copy(t_hbm.at[pl.ds(0, 8)], r0_buf, r_sem)
    r_copy.start()
    g_copy(0).start()
    g_copy(1).start()
    r_copy.wait()
    r0 = r0_buf[0]
    wb = w_ref[...].astype(jnp.bfloat16)

    for i in range(s_cnt):
        g_copy(i).wait()
        if i >= 2:
            for h in range(_N_HALF):
                o_copy(i - 2, h).wait()
        embb = _emb_step(x_ref[s_off + i], g_buf[i % 2], pe_ref[s_off + i],
                         r0, scale=scale)
        for h in range(_N_HALF):
            o_buf[i % 2, :, h * dh:(h + 1) * dh] = (
                jnp.dot(embb, wb[:, h * dh:(h + 1) * dh],
                        preferred_element_type=jnp.float32)
                + b_ref[:, h * dh:(h + 1) * dh]
            )
            o_copy(i, h).start()
        if i + 2 < s_cnt:
            g_copy(i + 2).start()
    for i in (s_cnt - 2, s_cnt - 1):
        for h in range(_N_HALF):
            o_copy(i, h).wait()


def _positional_encoding(seq_len, d):
    position = np.arange(seq_len, dtype=np.float32)[:, None]
    div_term = np.exp(
        np.arange(0, d, 2, dtype=np.float32) * (-np.log(10000.0) / d)
    )
    pe = np.zeros((seq_len, d), dtype=np.float32)
    pe[:, 0::2] = np.sin(position * div_term)
    pe[:, 1::2] = np.cos(position * div_term)
    return jnp.asarray(pe)


def _tc_body_acc(x_ref, g_hbm, pe_ref, w_ref, b_ref, t_hbm, prev_ref,
                 o_hbm, *scratch, scale, s_off, s_cnt):
    del prev_ref
    _tc_body(x_ref, g_hbm, pe_ref, w_ref, b_ref, t_hbm, o_hbm, *scratch,
             scale=scale, s_off=s_off, s_cnt=s_cnt)


_SLICES = (10, 10, 10, 10, 10)


def kernel(x, x_pad_mask, emb_table, proj_w, proj_b):
    seq, batch = x.shape
    vocab, emb_dim = emb_table.shape
    d_model = proj_w.shape[1]
    n_tok = seq * batch
    scale = math.sqrt(float(emb_dim))
    assert seq == sum(_SLICES)

    x = x.astype(jnp.int32)
    idx = x.reshape(n_tok)

    gathered = []
    off = 0
    for s_cnt in _SLICES:
        gathered.append(
            _make_sc_gather(vocab, emb_dim, n_tok, off * batch,
                            s_cnt * batch)(emb_table, idx)
        )
        off += s_cnt

    pe = _positional_encoding(seq, emb_dim).reshape(seq, 1, emb_dim)
    x3 = x.reshape(seq, batch // 128, 128)
    b2 = proj_b.reshape(1, d_model)

    out_shape = jax.ShapeDtypeStruct((seq, batch, d_model), jnp.float32)
    vmem = pl.BlockSpec(memory_space=pltpu.MemorySpace.VMEM)
    hbm = pl.BlockSpec(memory_space=pltpu.MemorySpace.HBM)
    scratch_shapes = [
        pltpu.VMEM((2, batch, emb_dim), jnp.float32),
        pltpu.VMEM((2, batch, d_model), jnp.float32),
        pltpu.VMEM((8, emb_dim), jnp.float32),
        pltpu.SemaphoreType.DMA((2,)),
        pltpu.SemaphoreType.DMA((2, _N_HALF)),
        pltpu.SemaphoreType.DMA,
    ]

    out = None
    off = 0
    for k, s_cnt in enumerate(_SLICES):
        g3 = gathered[k].reshape(s_cnt, batch, emb_dim)
        if out is None:
            out = pl.pallas_call(
                functools.partial(_tc_body, scale=scale, s_off=off,
                                  s_cnt=s_cnt),
                in_specs=[vmem, hbm, vmem, vmem, vmem, hbm],
                out_specs=hbm,
                out_shape=out_shape,
                scratch_shapes=scratch_shapes,
            )(x3, g3, pe, proj_w, b2, emb_table)
        else:
            out = pl.pallas_call(
                functools.partial(_tc_body_acc, scale=scale, s_off=off,
                                  s_cnt=s_cnt),
                in_specs=[vmem, hbm, vmem, vmem, vmem, hbm, hbm],
                out_specs=hbm,
                out_shape=out_shape,
                scratch_shapes=scratch_shapes,
                input_output_aliases={6: 0},
            )(x3, g3, pe, proj_w, b2, emb_table, out)
        off += s_cnt
    return out
